# baseline (speedup 1.0000x reference)
import functools

import jax
import jax.numpy as jnp
from jax.experimental import pallas as pl
from jax.experimental.pallas import tpu as pltpu


def _se_kernel(x_ref, w1_ref, b1_ref, w2_ref, b2_ref, o_ref, *, tb, c):
    x = x_ref[...]

    pooled_all = jnp.sum(x, axis=1, keepdims=True)

    if tb > 1:
        pooled = jnp.concatenate(
            [pooled_all[t * c:(t + 1) * c, :] for t in range(tb)], axis=1)
    else:
        pooled = pooled_all

    h = jnp.dot(w1_ref[...], pooled,
                preferred_element_type=jnp.float32) + b1_ref[...]
    h = jnp.maximum(h, 0.0)
    y = jnp.dot(w2_ref[...], h,
                preferred_element_type=jnp.float32) + b2_ref[...]
    gate = jax.nn.sigmoid(y)

    for t in range(tb):
        sl = pl.ds(t * c, c)
        o_ref[sl, :] = x[t * c:(t + 1) * c, :] * gate[:, t:t + 1]


def _pick_tb(b):
    for tb in (4, 2, 1):
        if b % tb == 0 and (b // tb) % 2 == 0:
            return tb
    return 1


@jax.jit
def _se_run(x, w1, b1, w2, b2):
    B, C, H, W = x.shape
    Cr = w1.shape[0]
    HW = H * W
    dsize = jnp.dtype(x.dtype).itemsize

    w1f = (w1 * (1.0 / HW)).astype(jnp.float32)
    w2f = w2.astype(jnp.float32)
    b1c = b1.reshape(Cr, 1).astype(jnp.float32)
    b2c = b2.reshape(C, 1).astype(jnp.float32)

    x2 = x.reshape(B * C, HW)

    TB = _pick_tb(B)
    n_steps = B // TB
    blk = TB * C

    cost = pl.CostEstimate(
        flops=int(4 * B * C * HW + 4 * B * C * Cr),
        transcendentals=int(B * C),
        bytes_accessed=int(2 * B * C * HW * dsize + 4 * (2 * C * Cr + C + Cr)),
    )

    out2 = pl.pallas_call(
        functools.partial(_se_kernel, tb=TB, c=C),
        out_shape=jax.ShapeDtypeStruct((B * C, HW), x.dtype),
        grid=(n_steps,),
        in_specs=[
            pl.BlockSpec((blk, HW), lambda i: (i, 0)),
            pl.BlockSpec((Cr, C), lambda i: (0, 0)),
            pl.BlockSpec((Cr, 1), lambda i: (0, 0)),
            pl.BlockSpec((C, Cr), lambda i: (0, 0)),
            pl.BlockSpec((C, 1), lambda i: (0, 0)),
        ],
        out_specs=pl.BlockSpec((blk, HW), lambda i: (i, 0)),
        compiler_params=pltpu.CompilerParams(
            dimension_semantics=("parallel",),
            vmem_limit_bytes=64 * 1024 * 1024),
        cost_estimate=cost,
    )(x2, w1f, b1c, w2f, b2c)

    return out2.reshape(B, C, H, W)


def kernel(x, w1, b1, w2, b2):
    return _se_run(x, w1, b1, w2, b2)


# speedup vs baseline: 6.9408x; 6.9408x over previous
import functools

import jax
import jax.numpy as jnp
from jax.experimental import pallas as pl
from jax.experimental.pallas import tpu as pltpu


def _se_kernel(x_ref, w1t_ref, b1_ref, w2t_ref, b2_ref, o_ref):
    x = x_ref[...]

    pooled = jnp.sum(x, axis=0)

    h = jnp.dot(pooled, w1t_ref[...],
                preferred_element_type=jnp.float32) + b1_ref[...]
    h = jnp.maximum(h, 0.0)
    y = jnp.dot(h, w2t_ref[...],
                preferred_element_type=jnp.float32) + b2_ref[...]
    gate = jax.nn.sigmoid(y)

    o_ref[...] = x * gate[None, :, :]


@jax.jit
def _se_run(x, w1, b1, w2, b2):
    B, C, H, W = x.shape
    Cr = w1.shape[0]
    HW = H * W
    dsize = jnp.dtype(x.dtype).itemsize

    w1t = (w1.T * (1.0 / HW)).astype(jnp.float32)
    w2t = w2.T.astype(jnp.float32)
    b1r = b1.reshape(1, Cr).astype(jnp.float32)
    b2r = b2.reshape(1, C).astype(jnp.float32)

    x_t = jnp.transpose(x, (2, 3, 0, 1)).reshape(HW, B, C)

    TBB = 8
    n_steps = B // TBB

    cost = pl.CostEstimate(
        flops=int(4 * B * C * HW + 4 * B * C * Cr),
        transcendentals=int(B * C),
        bytes_accessed=int(2 * B * C * HW * dsize + 4 * (2 * C * Cr + C + Cr)),
    )

    out_t = pl.pallas_call(
        _se_kernel,
        out_shape=jax.ShapeDtypeStruct((HW, B, C), x.dtype),
        grid=(n_steps,),
        in_specs=[
            pl.BlockSpec((HW, TBB, C), lambda i: (0, i, 0)),
            pl.BlockSpec((C, Cr), lambda i: (0, 0)),
            pl.BlockSpec((1, Cr), lambda i: (0, 0)),
            pl.BlockSpec((Cr, C), lambda i: (0, 0)),
            pl.BlockSpec((1, C), lambda i: (0, 0)),
        ],
        out_specs=pl.BlockSpec((HW, TBB, C), lambda i: (0, i, 0)),
        compiler_params=pltpu.CompilerParams(
            dimension_semantics=("parallel",),
            vmem_limit_bytes=64 * 1024 * 1024),
        cost_estimate=cost,
    )(x_t, w1t, b1r, w2t, b2r)

    return out_t.reshape(H, W, B, C).transpose(2, 3, 0, 1)


def kernel(x, w1, b1, w2, b2):
    return _se_run(x, w1, b1, w2, b2)
